# baseline (speedup 1.0000x reference)
import functools

import jax
import jax.numpy as jnp
from jax import lax
from jax.experimental import pallas as pl
from jax.experimental.pallas import tpu as pltpu
from jax.experimental.pallas import tpu_sc as plsc

DIM = 32
BATCH = 4096
HIST = 50
VOCAB = 1000000
VROWS = VOCAB - 1
NC, NS, L = 2, 16, 16
NW = NC * NS

WV = 768
NWIN = 999936 // WV
KMAIN = 40
KLAST = KMAIN * NW
TAIL0 = 999936
NTAIL = VROWS - TAIL0

BPW = BATCH // NW
BB = 16
NBLK = BPW // BB
NB = BB * HIST


def _repack_body(wt_hbm, wtail_hbm, s_hbm, chunk0, chunk1, obuf0, obuf1,
                 tbuf, zbuf, sem_in, sem_out):
    wid = lax.axis_index("s") * NC + lax.axis_index("c")
    lane = lax.iota(jnp.int32, L)
    lane32 = lane * DIM

    def start_in(k, buf):
        pltpu.async_copy(wt_hbm.at[:, pl.ds((k * NW + wid) * WV, WV)],
                         buf, sem_in)

    def wait_in():
        pltpu.make_async_copy(wt_hbm.at[:, pl.ds(0, WV)], chunk0,
                              sem_in).wait()

    def start_out(k, buf):
        pltpu.async_copy(
            buf, s_hbm.at[pl.ds(((k * NW + wid) * WV + 1) * DIM, WV * DIM)],
            sem_out)

    def wait_out():
        pltpu.make_async_copy(obuf0, s_hbm.at[pl.ds(DIM, WV * DIM)],
                              sem_out).wait()

    diag = [((lane + s) & (L - 1)) + dh * L for s in range(L) for dh in range(2)]

    def pack(buf, out):
        def body(c, carry2):
            cols = c * L + lane
            dstbase = cols * DIM
            for rows in diag:
                vals = plsc.load_gather(buf, [rows, cols])
                plsc.store_scatter(out, [dstbase + rows], vals)
            return carry2

        lax.fori_loop(0, WV // L, body, jnp.int32(0))

    start_in(0, chunk0)

    def step(jj, carry):
        k0 = 2 * jj
        k1 = 2 * jj + 1
        start_in(k1, chunk1)
        wait_in()

        @pl.when(jj >= 1)
        def _():
            wait_out()

        pack(chunk0, obuf0)
        start_out(k0, obuf0)

        @pl.when(jj + 1 < KMAIN // 2)
        def _():
            start_in(k0 + 2, chunk0)

        wait_in()

        @pl.when(jj >= 1)
        def _():
            wait_out()

        pack(chunk1, obuf1)
        start_out(k1, obuf1)
        return carry

    lax.fori_loop(0, KMAIN // 2, step, jnp.int32(0))
    wait_out()
    wait_out()

    @pl.when(wid < NWIN - KLAST)
    def _():
        pltpu.sync_copy(wt_hbm.at[:, pl.ds((KLAST + wid) * WV, WV)], chunk0)
        pack(chunk0, obuf0)
        pltpu.sync_copy(
            obuf0, s_hbm.at[pl.ds(((KLAST + wid) * WV + 1) * DIM, WV * DIM)])

    @pl.when(wid == 0)
    def _():
        zbuf[pl.ds(0, L)] = jnp.zeros((L,), jnp.float32)
        zbuf[pl.ds(L, L)] = jnp.zeros((L,), jnp.float32)
        pltpu.sync_copy(zbuf, s_hbm.at[pl.ds(0, DIM)])
        pltpu.sync_copy(wtail_hbm, tbuf)
        pltpu.sync_copy(tbuf, s_hbm.at[pl.ds((TAIL0 + 1) * DIM, NTAIL * DIM)])


def _gather_body(xt_hbm, w_hbm, out_hbm, xb, idxp, rows, sem):
    wid = lax.axis_index("s") * NC + lax.axis_index("c")
    bstart = wid * BPW

    pltpu.sync_copy(xt_hbm.at[:, pl.ds(bstart, BPW)], xb)

    lane = lax.iota(jnp.int32, L)

    for blk in range(NBLK):
        bl0 = blk * BB

        def transform(h, carry):
            v = xb[h, pl.ds(bl0, L)]
            rows_i = jnp.full((L,), bl0, jnp.int32) + lane
            cols_i = jnp.full((L,), h, jnp.int32)
            plsc.store_scatter(idxp, [rows_i, cols_i], v)
            return carry

        lax.fori_loop(0, HIST, transform, jnp.int32(0))

        cps = []
        for rb in range(BB):
            cps.append(pltpu.async_copy(
                w_hbm.at[idxp.at[bl0 + rb]],
                rows.at[rb],
                sem,
            ))
        for cp in cps:
            cp.wait()

        pltpu.sync_copy(rows, out_hbm.at[pl.ds(bstart + bl0, BB), :, :])


@functools.partial(jax.jit, static_argnums=())
def _embed(xt, wt, wtail):
    mesh = plsc.VectorSubcoreMesh(core_axis_name="c", subcore_axis_name="s")

    repack = functools.partial(
        pl.kernel,
        mesh=mesh,
        out_type=jax.ShapeDtypeStruct((VOCAB * DIM,), jnp.float32),
        scratch_types=[
            pltpu.VMEM((DIM, WV), jnp.float32),
            pltpu.VMEM((DIM, WV), jnp.float32),
            pltpu.VMEM((WV * DIM,), jnp.float32),
            pltpu.VMEM((WV * DIM,), jnp.float32),
            pltpu.VMEM((NTAIL * DIM,), jnp.float32),
            pltpu.VMEM((DIM,), jnp.float32),
            pltpu.SemaphoreType.DMA,
            pltpu.SemaphoreType.DMA,
        ],
        compiler_params=pltpu.CompilerParams(
            use_tc_tiling_on_sc=True, needs_layout_passes=False),
    )(_repack_body)
    s1d = repack(wt, wtail)
    w2d = s1d.reshape(VOCAB, DIM)

    gather = functools.partial(
        pl.kernel,
        mesh=mesh,
        out_type=jax.ShapeDtypeStruct((BATCH, HIST, DIM), jnp.float32),
        scratch_types=[
            pltpu.VMEM((HIST, BPW), jnp.int32),
            pltpu.VMEM((BPW, HIST), jnp.int32),
            pltpu.VMEM((BB, HIST, DIM), jnp.float32),
            pltpu.SemaphoreType.DMA,
        ],
        compiler_params=pltpu.CompilerParams(
            use_tc_tiling_on_sc=False, needs_layout_passes=False),
    )(_gather_body)
    return gather(xt, w2d)


def kernel(x, W):
    wtail = W[TAIL0:].reshape(-1)
    return _embed(x.T, W.T, wtail)


# speedup vs baseline: 1.5670x; 1.5670x over previous
import functools

import jax
import jax.numpy as jnp
from jax import lax
from jax.experimental import pallas as pl
from jax.experimental.pallas import tpu as pltpu
from jax.experimental.pallas import tpu_sc as plsc

DIM = 32
BATCH = 4096
HIST = 50
VOCAB = 1000000
VROWS = VOCAB - 1
NC, NS, L = 2, 16, 16
NW = NC * NS

WV = 768
NWIN = 999936 // WV
KMAIN = 40
KLAST = KMAIN * NW
TAIL0 = 999936
NTAIL = VROWS - TAIL0

BPW = BATCH // NW
BB = 16
NBLK = BPW // BB
NB = BB * HIST


def _repack_body(wt_hbm, wtail_hbm, s_hbm, chunk0, chunk1, obuf0, obuf1,
                 tbuf, zbuf, sem_in, sem_out):
    wid = lax.axis_index("s") * NC + lax.axis_index("c")
    lane = lax.iota(jnp.int32, L)
    lane32 = lane * DIM

    def start_in(k, buf):
        pltpu.async_copy(wt_hbm.at[:, pl.ds((k * NW + wid) * WV, WV)],
                         buf, sem_in)

    def wait_in():
        pltpu.make_async_copy(wt_hbm.at[:, pl.ds(0, WV)], chunk0,
                              sem_in).wait()

    def start_out(k, buf):
        pltpu.async_copy(
            buf, s_hbm.at[pl.ds(((k * NW + wid) * WV + 1) * DIM, WV * DIM)],
            sem_out)

    def wait_out():
        pltpu.make_async_copy(obuf0, s_hbm.at[pl.ds(DIM, WV * DIM)],
                              sem_out).wait()

    diag = [((lane + s) & (L - 1)) + dh * L for s in range(L) for dh in range(2)]

    def pack(buf, out):
        @plsc.parallel_loop(0, WV // L, unroll=2)
        def body(c):
            cols = c * L + lane
            dstbase = cols * DIM
            for rows in diag:
                vals = plsc.load_gather(buf, [rows, cols])
                plsc.store_scatter(out, [dstbase + rows], vals)

    start_in(0, chunk0)

    def step(jj, carry):
        k0 = 2 * jj
        k1 = 2 * jj + 1
        start_in(k1, chunk1)
        wait_in()

        @pl.when(jj >= 1)
        def _():
            wait_out()

        pack(chunk0, obuf0)
        start_out(k0, obuf0)

        @pl.when(jj + 1 < KMAIN // 2)
        def _():
            start_in(k0 + 2, chunk0)

        wait_in()

        @pl.when(jj >= 1)
        def _():
            wait_out()

        pack(chunk1, obuf1)
        start_out(k1, obuf1)
        return carry

    lax.fori_loop(0, KMAIN // 2, step, jnp.int32(0))
    wait_out()
    wait_out()

    @pl.when(wid < NWIN - KLAST)
    def _():
        pltpu.sync_copy(wt_hbm.at[:, pl.ds((KLAST + wid) * WV, WV)], chunk0)
        pack(chunk0, obuf0)
        pltpu.sync_copy(
            obuf0, s_hbm.at[pl.ds(((KLAST + wid) * WV + 1) * DIM, WV * DIM)])

    @pl.when(wid == 0)
    def _():
        zbuf[pl.ds(0, L)] = jnp.zeros((L,), jnp.float32)
        zbuf[pl.ds(L, L)] = jnp.zeros((L,), jnp.float32)
        pltpu.sync_copy(zbuf, s_hbm.at[pl.ds(0, DIM)])
        pltpu.sync_copy(wtail_hbm, tbuf)
        pltpu.sync_copy(tbuf, s_hbm.at[pl.ds((TAIL0 + 1) * DIM, NTAIL * DIM)])


def _gather_body(xt_hbm, w_hbm, out_hbm, xb, idxp, rows, sem):
    wid = lax.axis_index("s") * NC + lax.axis_index("c")
    bstart = wid * BPW

    pltpu.sync_copy(xt_hbm.at[:, pl.ds(bstart, BPW)], xb)

    lane = lax.iota(jnp.int32, L)

    for blk in range(NBLK):
        bl0 = blk * BB

        def transform(h, carry):
            v = xb[h, pl.ds(bl0, L)]
            rows_i = jnp.full((L,), bl0, jnp.int32) + lane
            cols_i = jnp.full((L,), h, jnp.int32)
            plsc.store_scatter(idxp, [rows_i, cols_i], v)
            return carry

        lax.fori_loop(0, HIST, transform, jnp.int32(0))

        cps = []
        for rb in range(BB):
            cps.append(pltpu.async_copy(
                w_hbm.at[idxp.at[bl0 + rb]],
                rows.at[rb],
                sem,
            ))
        for cp in cps:
            cp.wait()

        pltpu.sync_copy(rows, out_hbm.at[pl.ds(bstart + bl0, BB), :, :])


@functools.partial(jax.jit, static_argnums=())
def _embed(xt, wt, wtail):
    mesh = plsc.VectorSubcoreMesh(core_axis_name="c", subcore_axis_name="s")

    repack = functools.partial(
        pl.kernel,
        mesh=mesh,
        out_type=jax.ShapeDtypeStruct((VOCAB * DIM,), jnp.float32),
        scratch_types=[
            pltpu.VMEM((DIM, WV), jnp.float32),
            pltpu.VMEM((DIM, WV), jnp.float32),
            pltpu.VMEM((WV * DIM,), jnp.float32),
            pltpu.VMEM((WV * DIM,), jnp.float32),
            pltpu.VMEM((NTAIL * DIM,), jnp.float32),
            pltpu.VMEM((DIM,), jnp.float32),
            pltpu.SemaphoreType.DMA,
            pltpu.SemaphoreType.DMA,
        ],
        compiler_params=pltpu.CompilerParams(
            use_tc_tiling_on_sc=True, needs_layout_passes=False),
    )(_repack_body)
    s1d = repack(wt, wtail)
    w2d = s1d.reshape(VOCAB, DIM)

    gather = functools.partial(
        pl.kernel,
        mesh=mesh,
        out_type=jax.ShapeDtypeStruct((BATCH, HIST, DIM), jnp.float32),
        scratch_types=[
            pltpu.VMEM((HIST, BPW), jnp.int32),
            pltpu.VMEM((BPW, HIST), jnp.int32),
            pltpu.VMEM((BB, HIST, DIM), jnp.float32),
            pltpu.SemaphoreType.DMA,
        ],
        compiler_params=pltpu.CompilerParams(
            use_tc_tiling_on_sc=False, needs_layout_passes=False),
    )(_gather_body)
    return gather(xt, w2d)


def kernel(x, W):
    wtail = W[TAIL0:].reshape(-1)
    return _embed(x.T, W.T, wtail)


# speedup vs baseline: 1.5708x; 1.0024x over previous
import functools

import jax
import jax.numpy as jnp
from jax import lax
from jax.experimental import pallas as pl
from jax.experimental.pallas import tpu as pltpu
from jax.experimental.pallas import tpu_sc as plsc

DIM = 32
BATCH = 4096
HIST = 50
VOCAB = 1000000
VROWS = VOCAB - 1
NC, NS, L = 2, 16, 16
NW = NC * NS

WV = 768
NWIN = 999936 // WV
KMAIN = 40
KLAST = KMAIN * NW
TAIL0 = 999936
NTAIL = VROWS - TAIL0

BPW = BATCH // NW
BB = 16
NBLK = BPW // BB
NB = BB * HIST


def _repack_body(wt_hbm, wtail_hbm, s_hbm, chunk0, chunk1, obuf0, obuf1,
                 tbuf, zbuf, sem_in, sem_out):
    wid = lax.axis_index("s") * NC + lax.axis_index("c")
    lane = lax.iota(jnp.int32, L)
    lane32 = lane * DIM

    def start_in(k, buf):
        pltpu.async_copy(wt_hbm.at[:, pl.ds((k * NW + wid) * WV, WV)],
                         buf, sem_in)

    def wait_in():
        pltpu.make_async_copy(wt_hbm.at[:, pl.ds(0, WV)], chunk0,
                              sem_in).wait()

    def start_out(k, buf):
        pltpu.async_copy(
            buf, s_hbm.at[pl.ds(((k * NW + wid) * WV + 1) * DIM, WV * DIM)],
            sem_out)

    def wait_out():
        pltpu.make_async_copy(obuf0, s_hbm.at[pl.ds(DIM, WV * DIM)],
                              sem_out).wait()

    diag = [((lane + s) & (L - 1)) + dh * L for s in range(L) for dh in range(2)]

    def pack(buf, out):
        @plsc.parallel_loop(0, WV // L, unroll=2)
        def body(c):
            cols = c * L + lane
            dstbase = cols * DIM
            for rows in diag:
                vals = plsc.load_gather(buf, [rows, cols])
                plsc.store_scatter(out, [dstbase + rows], vals)

    start_in(0, chunk0)

    def step(jj, carry):
        k0 = 2 * jj
        k1 = 2 * jj + 1
        start_in(k1, chunk1)
        wait_in()

        @pl.when(jj >= 1)
        def _():
            wait_out()

        pack(chunk0, obuf0)
        start_out(k0, obuf0)

        @pl.when(jj + 1 < KMAIN // 2)
        def _():
            start_in(k0 + 2, chunk0)

        wait_in()

        @pl.when(jj >= 1)
        def _():
            wait_out()

        pack(chunk1, obuf1)
        start_out(k1, obuf1)
        return carry

    lax.fori_loop(0, KMAIN // 2, step, jnp.int32(0))
    wait_out()
    wait_out()

    @pl.when(wid < NWIN - KLAST)
    def _():
        pltpu.sync_copy(wt_hbm.at[:, pl.ds((KLAST + wid) * WV, WV)], chunk0)
        pack(chunk0, obuf0)
        pltpu.sync_copy(
            obuf0, s_hbm.at[pl.ds(((KLAST + wid) * WV + 1) * DIM, WV * DIM)])

    @pl.when(wid == 0)
    def _():
        zbuf[pl.ds(0, L)] = jnp.zeros((L,), jnp.float32)
        zbuf[pl.ds(L, L)] = jnp.zeros((L,), jnp.float32)
        pltpu.sync_copy(zbuf, s_hbm.at[pl.ds(0, DIM)])
        pltpu.sync_copy(wtail_hbm, tbuf)
        pltpu.sync_copy(tbuf, s_hbm.at[pl.ds((TAIL0 + 1) * DIM, NTAIL * DIM)])


def _gather_body(xt_hbm, w_hbm, out_hbm, xb, idxp, rows, sem):
    wid = lax.axis_index("s") * NC + lax.axis_index("c")
    bstart = wid * BPW

    pltpu.sync_copy(xt_hbm.at[:, pl.ds(bstart, BPW)], xb)

    lane = lax.iota(jnp.int32, L)

    for blk in range(NBLK):
        bl0 = blk * BB

        rows_i = jnp.full((L,), bl0, jnp.int32) + lane

        @plsc.parallel_loop(0, HIST, unroll=2)
        def transform(h):
            v = xb[h, pl.ds(bl0, L)]
            cols_i = jnp.full((L,), h, jnp.int32)
            plsc.store_scatter(idxp, [rows_i, cols_i], v)

        cps = []
        for rb in range(BB):
            cps.append(pltpu.async_copy(
                w_hbm.at[idxp.at[bl0 + rb]],
                rows.at[rb],
                sem,
            ))
        for cp in cps:
            cp.wait()

        pltpu.sync_copy(rows, out_hbm.at[pl.ds(bstart + bl0, BB), :, :])


@functools.partial(jax.jit, static_argnums=())
def _embed(xt, wt, wtail):
    mesh = plsc.VectorSubcoreMesh(core_axis_name="c", subcore_axis_name="s")

    repack = functools.partial(
        pl.kernel,
        mesh=mesh,
        out_type=jax.ShapeDtypeStruct((VOCAB * DIM,), jnp.float32),
        scratch_types=[
            pltpu.VMEM((DIM, WV), jnp.float32),
            pltpu.VMEM((DIM, WV), jnp.float32),
            pltpu.VMEM((WV * DIM,), jnp.float32),
            pltpu.VMEM((WV * DIM,), jnp.float32),
            pltpu.VMEM((NTAIL * DIM,), jnp.float32),
            pltpu.VMEM((DIM,), jnp.float32),
            pltpu.SemaphoreType.DMA,
            pltpu.SemaphoreType.DMA,
        ],
        compiler_params=pltpu.CompilerParams(
            use_tc_tiling_on_sc=True, needs_layout_passes=False),
    )(_repack_body)
    s1d = repack(wt, wtail)
    w2d = s1d.reshape(VOCAB, DIM)

    gather = functools.partial(
        pl.kernel,
        mesh=mesh,
        out_type=jax.ShapeDtypeStruct((BATCH, HIST, DIM), jnp.float32),
        scratch_types=[
            pltpu.VMEM((HIST, BPW), jnp.int32),
            pltpu.VMEM((BPW, HIST), jnp.int32),
            pltpu.VMEM((BB, HIST, DIM), jnp.float32),
            pltpu.SemaphoreType.DMA,
        ],
        compiler_params=pltpu.CompilerParams(
            use_tc_tiling_on_sc=False, needs_layout_passes=False),
    )(_gather_body)
    return gather(xt, w2d)


def kernel(x, W):
    wtail = W[TAIL0:].reshape(-1)
    return _embed(x.T, W.T, wtail)
